# baseline (device time: 16081 ns/iter reference)
import jax
import jax.numpy as jnp
from jax import lax
from jax.experimental import pallas as pl
from jax.experimental.pallas import tpu as pltpu

N_DEV = 8
N_LOCAL_E = 4
N_EXPERTS = 32
ROWS = 512
ROWS_PER = ROWS // N_DEV
D_OUT = 512


def kernel(x, router_W, route_idx, expert_W):
    def body(x_ref, rw_ref, idx_ref, ew_ref, out_ref,
             send_ref, recv_ref, send_sems, recv_sems):
        my = lax.axis_index("i")

        barrier_sem = pltpu.get_barrier_semaphore()
        for off in range(1, N_DEV):
            peer = (my + off) % N_DEV
            pl.semaphore_signal(
                barrier_sem, inc=1,
                device_id=peer, device_id_type=pl.DeviceIdType.LOGICAL,
            )
        pl.semaphore_wait(barrier_sem, N_DEV - 1)

        xf = x_ref[:, :]
        scores = jnp.dot(xf, rw_ref[:, :], preferred_element_type=jnp.float32)
        m = jnp.max(scores, axis=-1, keepdims=True)
        p = jnp.exp(scores - m)
        e_iota = lax.broadcasted_iota(jnp.int32, (ROWS, N_EXPERTS), 1)
        mask = (e_iota == idx_ref[:, 0:1]) | (e_iota == idx_ref[:, 1:2])
        gated = jnp.where(mask, p, 0.0)
        g = gated / jnp.sum(gated, axis=-1, keepdims=True)

        xb = xf.astype(jnp.bfloat16)
        gated_x = []
        for k in range(N_LOCAL_E):
            col = jnp.sum(
                jnp.where(e_iota == my * N_LOCAL_E + k, g, 0.0),
                axis=-1, keepdims=True,
            )
            gated_x.append(xb * col.astype(jnp.bfloat16))
        x_big = jnp.concatenate(gated_x, axis=1)
        w_big = ew_ref[:, :, :].astype(jnp.bfloat16).reshape(
            N_LOCAL_E * 256, D_OUT
        )
        acc = jnp.dot(x_big, w_big, preferred_element_type=jnp.float32)

        for j in range(N_DEV):
            send_ref[j, :, :] = acc[j * ROWS_PER:(j + 1) * ROWS_PER, :].astype(
                jnp.bfloat16
            )

        rdmas = []
        for off in range(1, N_DEV):
            dst = (my + off) % N_DEV
            rdma = pltpu.make_async_remote_copy(
                src_ref=send_ref.at[dst],
                dst_ref=recv_ref.at[off - 1],
                send_sem=send_sems.at[off - 1],
                recv_sem=recv_sems.at[off - 1],
                device_id=dst,
                device_id_type=pl.DeviceIdType.LOGICAL,
            )
            rdma.start()
            rdmas.append(rdma)

        own = send_ref[my, :, :].astype(jnp.float32)

        for rdma in rdmas:
            rdma.wait_recv()
        total = own
        for j in range(N_DEV - 1):
            total = total + recv_ref[j, :, :].astype(jnp.float32)
        out_ref[:, :] = total

        for rdma in rdmas:
            rdma.wait_send()

    return pl.pallas_call(
        body,
        out_shape=jax.ShapeDtypeStruct((ROWS_PER, D_OUT), jnp.float32),
        in_specs=[
            pl.BlockSpec(memory_space=pltpu.VMEM),
            pl.BlockSpec(memory_space=pltpu.VMEM),
            pl.BlockSpec(memory_space=pltpu.VMEM),
            pl.BlockSpec(memory_space=pltpu.VMEM),
        ],
        out_specs=pl.BlockSpec(memory_space=pltpu.VMEM),
        scratch_shapes=[
            pltpu.VMEM((N_DEV, ROWS_PER, D_OUT), jnp.bfloat16),
            pltpu.VMEM((N_DEV - 1, ROWS_PER, D_OUT), jnp.bfloat16),
            pltpu.SemaphoreType.DMA((N_DEV - 1,)),
            pltpu.SemaphoreType.DMA((N_DEV - 1,)),
        ],
        compiler_params=pltpu.CompilerParams(collective_id=0),
    )(x, router_W, route_idx, expert_W)


# device time: 7048 ns/iter; 2.2816x vs baseline; 2.2816x over previous
import jax
import jax.numpy as jnp
from jax import lax
from jax.experimental import pallas as pl
from jax.experimental.pallas import tpu as pltpu

import os
SKIP_COMM = os.environ.get("SKIP_COMM") == "1"
DIAG = os.environ.get("DIAG", "full")

N_DEV = 8
N_LOCAL_E = 4
N_EXPERTS = 32
ROWS = 512
ROWS_PER = ROWS // N_DEV
D_OUT = 512


def kernel(x, router_W, route_idx, expert_W):
    def body(x_ref, rw_ref, idx_ref, ew_ref, out_ref,
             send_ref, recv_ref, send_sems, recv_sems):
        my = lax.axis_index("i")

        if DIAG != "local":
            barrier_sem = pltpu.get_barrier_semaphore()
            for off in range(1, N_DEV):
                peer = (my + off) % N_DEV
                pl.semaphore_signal(
                    barrier_sem, inc=1,
                    device_id=peer, device_id_type=pl.DeviceIdType.LOGICAL,
                )
            pl.semaphore_wait(barrier_sem, N_DEV - 1)

        xf = x_ref[:, :]
        scores = jnp.dot(xf, rw_ref[:, :], preferred_element_type=jnp.float32)
        m = jnp.max(scores, axis=-1, keepdims=True)
        p = jnp.exp(scores - m)
        e_iota = lax.broadcasted_iota(jnp.int32, (ROWS, N_EXPERTS), 1)
        mask = (e_iota == idx_ref[:, 0:1]) | (e_iota == idx_ref[:, 1:2])
        gated = jnp.where(mask, p, 0.0)
        g = gated / jnp.sum(gated, axis=-1, keepdims=True)

        xb = xf.astype(jnp.bfloat16)
        if DIAG == "floor":
            acc = jnp.concatenate([xf, xf], axis=1)
        else:
            if DIAG == "nogates":
                gated_x = [xb] * N_LOCAL_E
            else:
                gated_x = []
                for k in range(N_LOCAL_E):
                    col = jnp.sum(
                        jnp.where(e_iota == my * N_LOCAL_E + k, g, 0.0),
                        axis=-1, keepdims=True,
                    )
                    gated_x.append(xb * col.astype(jnp.bfloat16))
            x_big = jnp.concatenate(gated_x, axis=1)
            w_big = ew_ref[:, :, :].astype(jnp.bfloat16).reshape(
                N_LOCAL_E * 256, D_OUT
            )
            acc = jnp.dot(x_big, w_big, preferred_element_type=jnp.float32)

        for j in range(N_DEV):
            send_ref[j, :, :] = acc[j * ROWS_PER:(j + 1) * ROWS_PER, :].astype(
                jnp.bfloat16
            )

        rdmas = []
        for off in range(1, N_DEV) if not SKIP_COMM else []:
            dst = (my + off) % N_DEV
            rdma = pltpu.make_async_remote_copy(
                src_ref=send_ref.at[dst],
                dst_ref=recv_ref.at[off - 1],
                send_sem=send_sems.at[off - 1],
                recv_sem=recv_sems.at[off - 1],
                device_id=dst,
                device_id_type=pl.DeviceIdType.LOGICAL,
            )
            rdma.start()
            rdmas.append(rdma)

        own = send_ref[my, :, :].astype(jnp.float32)

        for rdma in rdmas:
            rdma.wait_recv()
        total = own
        for j in range(N_DEV - 1):
            total = total + recv_ref[j, :, :].astype(jnp.float32)
        out_ref[:, :] = total

        for rdma in rdmas:
            rdma.wait_send()

    return pl.pallas_call(
        body,
        out_shape=jax.ShapeDtypeStruct((ROWS_PER, D_OUT), jnp.float32),
        in_specs=[
            pl.BlockSpec(memory_space=pltpu.VMEM),
            pl.BlockSpec(memory_space=pltpu.VMEM),
            pl.BlockSpec(memory_space=pltpu.VMEM),
            pl.BlockSpec(memory_space=pltpu.VMEM),
        ],
        out_specs=pl.BlockSpec(memory_space=pltpu.VMEM),
        scratch_shapes=[
            pltpu.VMEM((N_DEV, ROWS_PER, D_OUT), jnp.bfloat16),
            pltpu.VMEM((N_DEV - 1, ROWS_PER, D_OUT), jnp.bfloat16),
            pltpu.SemaphoreType.DMA((N_DEV - 1,)),
            pltpu.SemaphoreType.DMA((N_DEV - 1,)),
        ],
        compiler_params=(
            pltpu.CompilerParams()
            if DIAG == "local"
            else pltpu.CompilerParams(collective_id=0)
        ),
    )(x, router_W, route_idx, expert_W)
